# baseline (speedup 1.0000x reference)
import functools
import jax
import jax.numpy as jnp
from jax import lax
from jax.experimental import pallas as pl
from jax.experimental.pallas import tpu as pltpu
from jax.experimental.pallas import tpu_sc as plsc

_NC = 2
_NS = 16
_LANES = 16
_BB = 128


def _sc_counts_t(edges):
    B, A, D = edges.shape
    NW = _NC * _NS
    LPW = B // NW
    NBLK = B // _BB
    AC = 16
    assert D == _LANES and A % AC == 0 and LPW == 32

    mesh = plsc.VectorSubcoreMesh(core_axis_name="c", subcore_axis_name="s")

    @functools.partial(
        pl.kernel,
        out_type=jax.ShapeDtypeStruct((NBLK, A, A, _BB), jnp.float32),
        mesh=mesh,
        scratch_types=[
            pltpu.VMEM((LPW, A, D), jnp.int32),
            pltpu.VMEM((AC, A, LPW), jnp.float32),
        ],
        compiler_params=pltpu.CompilerParams(
            needs_layout_passes=False, use_tc_tiling_on_sc=False),
    )
    def k(edges_hbm, n_hbm, edges_v, n_v):
        wid = lax.axis_index("s") * _NC + lax.axis_index("c")
        b0 = wid * LPW
        blk = b0 // _BB
        l0 = b0 % _BB
        ones = jnp.full((_LANES,), 1.0, jnp.float32)
        zeros = jnp.zeros((_LANES,), jnp.float32)
        iota = lax.iota(jnp.int32, _LANES)
        bvecs = [iota + h * _LANES for h in range(LPW // _LANES)]

        pltpu.sync_copy(edges_hbm.at[pl.ds(b0, LPW)], edges_v)

        def chunk_body(ci, _):
            a0 = ci * AC

            def zero_body(i, _):
                for h in range(LPW // _LANES):
                    n_v[i // A, i % A, pl.ds(h * _LANES, _LANES)] = zeros
                return 0

            lax.fori_loop(0, AC * A, zero_body, 0, unroll=8)

            def a_body(al, _):
                a_vec = jnp.full((_LANES,), al, jnp.int32)
                ag_vec = jnp.full((_LANES,), a0 + al, jnp.int32)
                for h in range(LPW // _LANES):
                    b_vec = bvecs[h]
                    for d in range(D):
                        e = plsc.load_gather(
                            edges_v,
                            [b_vec, ag_vec, jnp.full((_LANES,), d, jnp.int32)])
                        plsc.addupdate_scatter(
                            n_v, [a_vec, e, b_vec], ones)
                return 0

            lax.fori_loop(0, AC, a_body, 0)
            pltpu.sync_copy(
                n_v, n_hbm.at[blk, pl.ds(a0, AC), :, pl.ds(l0, LPW)])
            return 0

        lax.fori_loop(0, A // AC, chunk_body, 0)

    return k(edges)


def _tc_dense_t(n4, atoms_t, bonds_t, w_aT, w_bT):
    A, FA, B = atoms_t.shape
    D, FB = bonds_t.shape[1], bonds_t.shape[2]
    C = w_aT.shape[0]
    NBLK = B // _BB
    AC = 4

    def body(n_ref, atoms_ref, bonds_ref, wa_ref, wb_ref, out_ref, p_scr):
        waT = wa_ref[...]
        wbT = wb_ref[...]
        for a in range(A):
            p_scr[a] = jnp.dot(waT, atoms_ref[a],
                               preferred_element_type=jnp.float32)
        for a0 in range(0, A, AC):
            hs = []
            for i in range(AC):
                sb = jnp.sum(bonds_ref[a0 + i], axis=0)
                r = jnp.dot(wbT, sb, preferred_element_type=jnp.float32)
                hs.append(r + p_scr[a0 + i])

            def jbody(j, hs):
                pj = p_scr[j]
                return tuple(
                    hs[i] + pj * n_ref[0, a0 + i, j, :][None, :]
                    for i in range(AC))

            hs = lax.fori_loop(0, A, jbody, tuple(hs))
            for i in range(AC):
                h = hs[i]
                out_ref[a0 + i] = jnp.where(
                    h > 0, h, jnp.exp(jnp.minimum(h, 0.0)) - 1.0)

    return pl.pallas_call(
        body,
        grid=(NBLK,),
        in_specs=[
            pl.BlockSpec((1, A, A, _BB), lambda i: (i, 0, 0, 0)),
            pl.BlockSpec((A, FA, _BB), lambda i: (0, 0, i)),
            pl.BlockSpec((A, D, FB, _BB), lambda i: (0, 0, 0, i)),
            pl.BlockSpec((C, FA), lambda i: (0, 0)),
            pl.BlockSpec((C, FB), lambda i: (0, 0)),
        ],
        out_specs=pl.BlockSpec((A, C, _BB), lambda i: (0, 0, i)),
        out_shape=jax.ShapeDtypeStruct((A, C, B), jnp.float32),
        scratch_shapes=[pltpu.VMEM((A, C, _BB), jnp.float32)],
    )(n4, atoms_t, bonds_t, w_aT, w_bT)


def kernel(atoms, bonds, edges, W):
    B, A, FA = atoms.shape
    D = edges.shape[-1]
    wd = W[D]
    w_aT = wd[:FA].T
    w_bT = wd[FA:].T
    atoms_t = atoms.transpose(1, 2, 0)
    bonds_t = bonds.transpose(1, 2, 3, 0)
    n4 = _sc_counts_t(edges)
    out_t = _tc_dense_t(n4, atoms_t, bonds_t, w_aT, w_bT)
    return out_t.transpose(2, 0, 1)


# speedup vs baseline: 1.2788x; 1.2788x over previous
import functools
import jax
import jax.numpy as jnp
from jax import lax
from jax.experimental import pallas as pl
from jax.experimental.pallas import tpu as pltpu
from jax.experimental.pallas import tpu_sc as plsc

_NC = 2
_NS = 16
_LANES = 16
_BB = 128


def _sc_counts_t(edges):
    B, A, D = edges.shape
    NW = _NC * _NS
    LPW = B // NW
    NBLK = B // _BB
    AC = 16
    assert D == _LANES and A % AC == 0 and LPW == 32

    mesh = plsc.VectorSubcoreMesh(core_axis_name="c", subcore_axis_name="s")

    @functools.partial(
        pl.kernel,
        out_type=jax.ShapeDtypeStruct((NBLK, A, A, _BB), jnp.float32),
        mesh=mesh,
        scratch_types=[
            pltpu.VMEM((LPW, A, D), jnp.int32),
            pltpu.VMEM((AC, A, LPW), jnp.float32),
        ],
        compiler_params=pltpu.CompilerParams(
            needs_layout_passes=False, use_tc_tiling_on_sc=False),
    )
    def k(edges_hbm, n_hbm, edges_v, n_v):
        wid = lax.axis_index("s") * _NC + lax.axis_index("c")
        b0 = wid * LPW
        blk = b0 // _BB
        l0 = b0 % _BB
        ones = jnp.full((_LANES,), 1.0, jnp.float32)
        zeros = jnp.zeros((_LANES,), jnp.float32)
        iota = lax.iota(jnp.int32, _LANES)
        bvecs = [iota + h * _LANES for h in range(LPW // _LANES)]

        pltpu.sync_copy(edges_hbm.at[pl.ds(b0, LPW)], edges_v)

        def chunk_body(ci, _):
            a0 = ci * AC

            def zero_body(i, _):
                for h in range(LPW // _LANES):
                    n_v[i // A, i % A, pl.ds(h * _LANES, _LANES)] = zeros
                return 0

            lax.fori_loop(0, AC * A, zero_body, 0, unroll=8)

            def a_body(al, _):
                a_vec = jnp.full((_LANES,), al, jnp.int32)
                ag_vec = jnp.full((_LANES,), a0 + al, jnp.int32)
                for h in range(LPW // _LANES):
                    b_vec = bvecs[h]
                    for d in range(D):
                        e = plsc.load_gather(
                            edges_v,
                            [b_vec, ag_vec, jnp.full((_LANES,), d, jnp.int32)])
                        plsc.addupdate_scatter(
                            n_v, [a_vec, e, b_vec], ones)
                return 0

            lax.fori_loop(0, AC, a_body, 0)
            pltpu.sync_copy(
                n_v, n_hbm.at[blk, pl.ds(a0, AC), :, pl.ds(l0, LPW)])
            return 0

        lax.fori_loop(0, A // AC, chunk_body, 0)

    return k(edges)


def _tc_dense_t(n4, atoms_t, bonds_t, w_aT, w_bT):
    A, FA, B = atoms_t.shape
    D, FB = bonds_t.shape[1], bonds_t.shape[2]
    C = w_aT.shape[0]
    NBLK = B // _BB
    AC = 8
    CH = C // 2

    def body(n_ref, atoms_ref, bonds_ref, wa_ref, wb_ref, out_ref,
             p_scr, r_scr):
        waT = wa_ref[...]
        wbT = wb_ref[...]
        for a in range(A):
            p_scr[a] = jnp.dot(waT, atoms_ref[a],
                               preferred_element_type=jnp.float32)
        for a in range(A):
            sb = jnp.sum(bonds_ref[a], axis=0)
            r = jnp.dot(wbT, sb, preferred_element_type=jnp.float32)
            r_scr[a] = r + p_scr[a]
        for ch in range(C // CH):
            c0 = ch * CH
            for a0 in range(0, A, AC):
                hs = tuple(r_scr[a0 + i, c0:c0 + CH, :] for i in range(AC))

                def jbody(j, hs):
                    pj = p_scr[j, c0:c0 + CH, :]
                    return tuple(
                        hs[i] + pj * n_ref[0, a0 + i, j, :][None, :]
                        for i in range(AC))

                hs = lax.fori_loop(0, A, jbody, hs, unroll=2)
                for i in range(AC):
                    h = hs[i]
                    out_ref[a0 + i, c0:c0 + CH, :] = jnp.where(
                        h > 0, h, jnp.exp(jnp.minimum(h, 0.0)) - 1.0)

    return pl.pallas_call(
        body,
        grid=(NBLK,),
        in_specs=[
            pl.BlockSpec((1, A, A, _BB), lambda i: (i, 0, 0, 0)),
            pl.BlockSpec((A, FA, _BB), lambda i: (0, 0, i)),
            pl.BlockSpec((A, D, FB, _BB), lambda i: (0, 0, 0, i)),
            pl.BlockSpec((C, FA), lambda i: (0, 0)),
            pl.BlockSpec((C, FB), lambda i: (0, 0)),
        ],
        out_specs=pl.BlockSpec((A, C, _BB), lambda i: (0, 0, i)),
        out_shape=jax.ShapeDtypeStruct((A, C, B), jnp.float32),
        scratch_shapes=[pltpu.VMEM((A, C, _BB), jnp.float32),
                        pltpu.VMEM((A, C, _BB), jnp.float32)],
    )(n4, atoms_t, bonds_t, w_aT, w_bT)


def kernel(atoms, bonds, edges, W):
    B, A, FA = atoms.shape
    D = edges.shape[-1]
    wd = W[D]
    w_aT = wd[:FA].T
    w_bT = wd[FA:].T
    atoms_t = atoms.transpose(1, 2, 0)
    bonds_t = bonds.transpose(1, 2, 3, 0)
    n4 = _sc_counts_t(edges)
    out_t = _tc_dense_t(n4, atoms_t, bonds_t, w_aT, w_bT)
    return out_t.transpose(2, 0, 1)


# speedup vs baseline: 1.3715x; 1.0725x over previous
import functools
import jax
import jax.numpy as jnp
from jax import lax
from jax.experimental import pallas as pl
from jax.experimental.pallas import tpu as pltpu
from jax.experimental.pallas import tpu_sc as plsc

_NC = 2
_NS = 16
_LANES = 16
_BB = 128


def _sc_counts_t(edges):
    B, A, D = edges.shape
    NW = _NC * _NS
    LPW = B // NW
    NBLK = B // _BB
    AC = 16
    assert D == _LANES and A % AC == 0 and LPW == 32

    mesh = plsc.VectorSubcoreMesh(core_axis_name="c", subcore_axis_name="s")

    @functools.partial(
        pl.kernel,
        out_type=jax.ShapeDtypeStruct((NBLK, A, A, _BB), jnp.float32),
        mesh=mesh,
        scratch_types=[
            pltpu.VMEM((LPW, A, D), jnp.int32),
            pltpu.VMEM((AC, A, LPW), jnp.float32),
        ],
        compiler_params=pltpu.CompilerParams(
            needs_layout_passes=False, use_tc_tiling_on_sc=False),
    )
    def k(edges_hbm, n_hbm, edges_v, n_v):
        wid = lax.axis_index("s") * _NC + lax.axis_index("c")
        b0 = wid * LPW
        blk = b0 // _BB
        l0 = b0 % _BB
        ones = jnp.full((_LANES,), 1.0, jnp.float32)
        zeros = jnp.zeros((_LANES,), jnp.float32)
        iota = lax.iota(jnp.int32, _LANES)
        bvecs = [iota + h * _LANES for h in range(LPW // _LANES)]

        pltpu.sync_copy(edges_hbm.at[pl.ds(b0, LPW)], edges_v)

        def chunk_body(ci, _):
            a0 = ci * AC

            def zero_body(i, _):
                for h in range(LPW // _LANES):
                    n_v[i // A, i % A, pl.ds(h * _LANES, _LANES)] = zeros
                return 0

            lax.fori_loop(0, AC * A, zero_body, 0, unroll=8)

            def a_body(al, _):
                a_vec = jnp.full((_LANES,), al, jnp.int32)
                ag_vec = jnp.full((_LANES,), a0 + al, jnp.int32)
                for h in range(LPW // _LANES):
                    b_vec = bvecs[h]
                    for d in range(D):
                        e = plsc.load_gather(
                            edges_v,
                            [b_vec, ag_vec, jnp.full((_LANES,), d, jnp.int32)])
                        plsc.addupdate_scatter(
                            n_v, [a_vec, e, b_vec], ones)
                return 0

            lax.fori_loop(0, AC, a_body, 0)
            pltpu.sync_copy(
                n_v, n_hbm.at[blk, pl.ds(a0, AC), :, pl.ds(l0, LPW)])
            return 0

        lax.fori_loop(0, A // AC, chunk_body, 0)

    return k(edges)


def _tc_dense_t(n4, atoms_t, bonds_t, w_aT, w_bT):
    A, FA, B = atoms_t.shape
    D, FB = bonds_t.shape[1], bonds_t.shape[2]
    C = w_aT.shape[0]
    NBLK = B // _BB
    AC = 8
    CH = C // 2

    def body(n_ref, atoms_ref, bonds_ref, wa_ref, wb_ref, out_ref,
             p_scr, r_scr):
        waT = wa_ref[...].astype(jnp.bfloat16)
        wbT = wb_ref[...]
        for a in range(A):
            p_scr[a] = jnp.dot(waT, atoms_ref[a].astype(jnp.bfloat16),
                               preferred_element_type=jnp.float32)
        for a in range(A):
            sb = jnp.sum(bonds_ref[a], axis=0)
            r = jnp.dot(wbT, sb, preferred_element_type=jnp.float32)
            r_scr[a] = r + p_scr[a]
        for ch in range(C // CH):
            c0 = ch * CH
            for a0 in range(0, A, AC):
                hs = tuple(r_scr[a0 + i, c0:c0 + CH, :] for i in range(AC))

                def jbody(j, hs):
                    pj = p_scr[j, c0:c0 + CH, :]
                    return tuple(
                        hs[i] + pj * n_ref[0, a0 + i, j, :][None, :]
                        for i in range(AC))

                hs = lax.fori_loop(0, A, jbody, hs, unroll=4)
                for i in range(AC):
                    h = hs[i]
                    out_ref[a0 + i, c0:c0 + CH, :] = jnp.where(
                        h > 0, h, jnp.exp(jnp.minimum(h, 0.0)) - 1.0)

    return pl.pallas_call(
        body,
        grid=(NBLK,),
        in_specs=[
            pl.BlockSpec((1, A, A, _BB), lambda i: (i, 0, 0, 0)),
            pl.BlockSpec((A, FA, _BB), lambda i: (0, 0, i)),
            pl.BlockSpec((A, D, FB, _BB), lambda i: (0, 0, 0, i)),
            pl.BlockSpec((C, FA), lambda i: (0, 0)),
            pl.BlockSpec((C, FB), lambda i: (0, 0)),
        ],
        out_specs=pl.BlockSpec((A, C, _BB), lambda i: (0, 0, i)),
        out_shape=jax.ShapeDtypeStruct((A, C, B), jnp.float32),
        scratch_shapes=[pltpu.VMEM((A, C, _BB), jnp.float32),
                        pltpu.VMEM((A, C, _BB), jnp.float32)],
    )(n4, atoms_t, bonds_t, w_aT, w_bT)


def kernel(atoms, bonds, edges, W):
    B, A, FA = atoms.shape
    D = edges.shape[-1]
    wd = W[D]
    w_aT = wd[:FA].T
    w_bT = wd[FA:].T
    atoms_t = atoms.transpose(1, 2, 0)
    bonds_t = bonds.transpose(1, 2, 3, 0)
    n4 = _sc_counts_t(edges)
    out_t = _tc_dense_t(n4, atoms_t, bonds_t, w_aT, w_bT)
    return out_t.transpose(2, 0, 1)


# speedup vs baseline: 1.3904x; 1.0138x over previous
import functools
import jax
import jax.numpy as jnp
from jax import lax
from jax.experimental import pallas as pl
from jax.experimental.pallas import tpu as pltpu
from jax.experimental.pallas import tpu_sc as plsc

_NC = 2
_NS = 16
_LANES = 16
_BB = 128


def _sc_counts_t(edges):
    B, A, D = edges.shape
    NW = _NC * _NS
    LPW = B // NW
    NBLK = B // _BB
    AC = 16
    assert D == _LANES and A % AC == 0 and LPW == 32

    mesh = plsc.VectorSubcoreMesh(core_axis_name="c", subcore_axis_name="s")

    @functools.partial(
        pl.kernel,
        out_type=jax.ShapeDtypeStruct((NBLK, A, A, _BB), jnp.float32),
        mesh=mesh,
        scratch_types=[
            pltpu.VMEM((LPW, A, D), jnp.int32),
            pltpu.VMEM((AC, A, LPW), jnp.float32),
            pltpu.VMEM((AC, A, LPW), jnp.float32),
            pltpu.SemaphoreType.DMA,
            pltpu.SemaphoreType.DMA,
        ],
        compiler_params=pltpu.CompilerParams(
            needs_layout_passes=False, use_tc_tiling_on_sc=False),
    )
    def k(edges_hbm, n_hbm, edges_v, n_v0, n_v1, sem0, sem1):
        wid = lax.axis_index("s") * _NC + lax.axis_index("c")
        b0 = wid * LPW
        blk = b0 // _BB
        l0 = b0 % _BB
        ones = jnp.full((_LANES,), 1.0, jnp.float32)
        zeros = jnp.zeros((_LANES,), jnp.float32)
        iota = lax.iota(jnp.int32, _LANES)
        bvecs = [iota + h * _LANES for h in range(LPW // _LANES)]

        pltpu.sync_copy(edges_hbm.at[pl.ds(b0, LPW)], edges_v)

        bufs = (n_v0, n_v1)
        sems = (sem0, sem1)
        copies = [None, None]
        for ci in range(A // AC):
            a0 = ci * AC
            n_v = bufs[ci % 2]
            if copies[ci % 2] is not None:
                copies[ci % 2].wait()

            def zero_body(i, _, n_v=n_v):
                for h in range(LPW // _LANES):
                    n_v[i // A, i % A, pl.ds(h * _LANES, _LANES)] = zeros
                return 0

            lax.fori_loop(0, AC * A, zero_body, 0, unroll=8)

            def a_body(al, _, n_v=n_v, a0=a0):
                a_vec = jnp.full((_LANES,), al, jnp.int32)
                ag_vec = jnp.full((_LANES,), a0 + al, jnp.int32)
                for h in range(LPW // _LANES):
                    b_vec = bvecs[h]
                    for d in range(D):
                        e = plsc.load_gather(
                            edges_v,
                            [b_vec, ag_vec, jnp.full((_LANES,), d, jnp.int32)])
                        plsc.addupdate_scatter(
                            n_v, [a_vec, e, b_vec], ones)
                return 0

            lax.fori_loop(0, AC, a_body, 0)
            copies[ci % 2] = pltpu.make_async_copy(
                n_v, n_hbm.at[blk, pl.ds(a0, AC), :, pl.ds(l0, LPW)],
                sems[ci % 2])
            copies[ci % 2].start()
        for cp in copies:
            if cp is not None:
                cp.wait()

    return k(edges)


def _tc_dense_t(n4, atoms_t, bonds_t, w_aT, w_bT):
    A, FA, B = atoms_t.shape
    D, FB = bonds_t.shape[1], bonds_t.shape[2]
    C = w_aT.shape[0]
    NBLK = B // _BB
    AC = 8
    CH = C // 2

    def body(n_ref, atoms_ref, bonds_ref, wa_ref, wb_ref, out_ref,
             p_scr, r_scr):
        waT = wa_ref[...].astype(jnp.bfloat16)
        wbT = wb_ref[...]
        for a in range(A):
            p_scr[a] = jnp.dot(waT, atoms_ref[a].astype(jnp.bfloat16),
                               preferred_element_type=jnp.float32)
        for a in range(A):
            sb = jnp.sum(bonds_ref[a], axis=0)
            r = jnp.dot(wbT, sb, preferred_element_type=jnp.float32)
            r_scr[a] = r + p_scr[a]
        for ch in range(C // CH):
            c0 = ch * CH
            for a0 in range(0, A, AC):
                hs = tuple(r_scr[a0 + i, c0:c0 + CH, :] for i in range(AC))

                def jbody(j, hs):
                    pj = p_scr[j, c0:c0 + CH, :]
                    return tuple(
                        hs[i] + pj * n_ref[0, a0 + i, j, :][None, :]
                        for i in range(AC))

                hs = lax.fori_loop(0, A, jbody, hs, unroll=4)
                for i in range(AC):
                    h = hs[i]
                    out_ref[a0 + i, c0:c0 + CH, :] = jnp.where(
                        h > 0, h, jnp.exp(jnp.minimum(h, 0.0)) - 1.0)

    return pl.pallas_call(
        body,
        grid=(NBLK,),
        in_specs=[
            pl.BlockSpec((1, A, A, _BB), lambda i: (i, 0, 0, 0)),
            pl.BlockSpec((A, FA, _BB), lambda i: (0, 0, i)),
            pl.BlockSpec((A, D, FB, _BB), lambda i: (0, 0, 0, i)),
            pl.BlockSpec((C, FA), lambda i: (0, 0)),
            pl.BlockSpec((C, FB), lambda i: (0, 0)),
        ],
        out_specs=pl.BlockSpec((A, C, _BB), lambda i: (0, 0, i)),
        out_shape=jax.ShapeDtypeStruct((A, C, B), jnp.float32),
        scratch_shapes=[pltpu.VMEM((A, C, _BB), jnp.float32),
                        pltpu.VMEM((A, C, _BB), jnp.float32)],
    )(n4, atoms_t, bonds_t, w_aT, w_bT)


def kernel(atoms, bonds, edges, W):
    B, A, FA = atoms.shape
    D = edges.shape[-1]
    wd = W[D]
    w_aT = wd[:FA].T
    w_bT = wd[FA:].T
    atoms_t = atoms.transpose(1, 2, 0)
    bonds_t = bonds.transpose(1, 2, 3, 0)
    n4 = _sc_counts_t(edges)
    out_t = _tc_dense_t(n4, atoms_t, bonds_t, w_aT, w_bT)
    return out_t.transpose(2, 0, 1)


# speedup vs baseline: 1.4791x; 1.0638x over previous
import functools
import jax
import jax.numpy as jnp
from jax import lax
from jax.experimental import pallas as pl
from jax.experimental.pallas import tpu as pltpu
from jax.experimental.pallas import tpu_sc as plsc

_NC = 2
_NS = 16
_LANES = 16
_BB = 128


def _sc_counts_t(edges):
    B, A, D = edges.shape
    NW = _NC * _NS
    LPW = B // NW
    NBLK = B // _BB
    AC = 16
    assert D == _LANES and A % AC == 0 and LPW == 32

    mesh = plsc.VectorSubcoreMesh(core_axis_name="c", subcore_axis_name="s")

    @functools.partial(
        pl.kernel,
        out_type=jax.ShapeDtypeStruct((NBLK, A, A, _BB), jnp.float32),
        mesh=mesh,
        scratch_types=[
            pltpu.VMEM((LPW, A, D), jnp.int32),
            pltpu.VMEM((AC, A, LPW), jnp.float32),
            pltpu.VMEM((AC, A, LPW), jnp.float32),
            pltpu.SemaphoreType.DMA,
            pltpu.SemaphoreType.DMA,
        ],
        compiler_params=pltpu.CompilerParams(
            needs_layout_passes=False, use_tc_tiling_on_sc=False),
    )
    def k(edges_hbm, n_hbm, edges_v, n_v0, n_v1, sem0, sem1):
        wid = lax.axis_index("s") * _NC + lax.axis_index("c")
        b0 = wid * LPW
        blk = b0 // _BB
        l0 = b0 % _BB
        ones = jnp.full((_LANES,), 1.0, jnp.float32)
        zeros = jnp.zeros((_LANES,), jnp.float32)
        iota = lax.iota(jnp.int32, _LANES)
        bvecs = [iota + h * _LANES for h in range(LPW // _LANES)]

        pltpu.sync_copy(edges_hbm.at[pl.ds(b0, LPW)], edges_v)

        bufs = (n_v0, n_v1)
        sems = (sem0, sem1)
        copies = [None, None]
        for ci in range(A // AC):
            a0 = ci * AC
            n_v = bufs[ci % 2]
            if copies[ci % 2] is not None:
                copies[ci % 2].wait()

            def zero_body(i, _, n_v=n_v):
                for h in range(LPW // _LANES):
                    n_v[i // A, i % A, pl.ds(h * _LANES, _LANES)] = zeros
                return 0

            lax.fori_loop(0, AC * A, zero_body, 0, unroll=8)

            def a_body(al, _, n_v=n_v, a0=a0):
                a_vec = jnp.full((_LANES,), al, jnp.int32)
                ag_vec = jnp.full((_LANES,), a0 + al, jnp.int32)
                for h in range(LPW // _LANES):
                    b_vec = bvecs[h]
                    for d in range(D):
                        e = plsc.load_gather(
                            edges_v,
                            [b_vec, ag_vec, jnp.full((_LANES,), d, jnp.int32)])
                        plsc.addupdate_scatter(
                            n_v, [a_vec, e, b_vec], ones)
                return 0

            lax.fori_loop(0, AC, a_body, 0)
            copies[ci % 2] = pltpu.make_async_copy(
                n_v, n_hbm.at[blk, pl.ds(a0, AC), :, pl.ds(l0, LPW)],
                sems[ci % 2])
            copies[ci % 2].start()
        for cp in copies:
            if cp is not None:
                cp.wait()

    return k(edges)


def _tc_dense_t(n4, atoms_t, bonds_t, w_aT, w_bT):
    A, FA, B = atoms_t.shape
    D, FB = bonds_t.shape[1], bonds_t.shape[2]
    C = w_aT.shape[0]
    NBLK = B // _BB
    AC = 8
    CH = C // 2

    def body(n_ref, atoms_ref, bonds_ref, wa_ref, wb_ref, out_ref,
             p_scr, r_scr):
        waT = wa_ref[...].astype(jnp.bfloat16)
        wbT = wb_ref[...]
        for a in range(A):
            p_scr[a] = jnp.dot(waT, atoms_ref[a].astype(jnp.bfloat16),
                               preferred_element_type=jnp.float32)
        for a in range(A):
            sb = jnp.sum(bonds_ref[a], axis=0)
            r = jnp.dot(wbT, sb, preferred_element_type=jnp.float32)
            r_scr[a] = r + p_scr[a]
        for ch in range(C // CH):
            c0 = ch * CH
            for a0 in range(0, A, AC):
                hs = tuple(r_scr[a0 + i, c0:c0 + CH, :] for i in range(AC))

                def jbody(j, hs):
                    pj = p_scr[j, c0:c0 + CH, :]
                    nb = n_ref[0, a0:a0 + AC, j, :]
                    return tuple(
                        hs[i] + pj * nb[i:i + 1, :]
                        for i in range(AC))

                hs = lax.fori_loop(0, A, jbody, hs, unroll=8)
                for i in range(AC):
                    h = hs[i]
                    out_ref[a0 + i, c0:c0 + CH, :] = jnp.where(
                        h > 0, h, jnp.exp(jnp.minimum(h, 0.0)) - 1.0)

    return pl.pallas_call(
        body,
        grid=(NBLK,),
        in_specs=[
            pl.BlockSpec((1, A, A, _BB), lambda i: (i, 0, 0, 0)),
            pl.BlockSpec((A, FA, _BB), lambda i: (0, 0, i)),
            pl.BlockSpec((A, D, FB, _BB), lambda i: (0, 0, 0, i)),
            pl.BlockSpec((C, FA), lambda i: (0, 0)),
            pl.BlockSpec((C, FB), lambda i: (0, 0)),
        ],
        out_specs=pl.BlockSpec((A, C, _BB), lambda i: (0, 0, i)),
        out_shape=jax.ShapeDtypeStruct((A, C, B), jnp.float32),
        scratch_shapes=[pltpu.VMEM((A, C, _BB), jnp.float32),
                        pltpu.VMEM((A, C, _BB), jnp.float32)],
    )(n4, atoms_t, bonds_t, w_aT, w_bT)


def kernel(atoms, bonds, edges, W):
    B, A, FA = atoms.shape
    D = edges.shape[-1]
    wd = W[D]
    w_aT = wd[:FA].T
    w_bT = wd[FA:].T
    atoms_t = atoms.transpose(1, 2, 0)
    bonds_t = bonds.transpose(1, 2, 3, 0)
    n4 = _sc_counts_t(edges)
    out_t = _tc_dense_t(n4, atoms_t, bonds_t, w_aT, w_bT)
    return out_t.transpose(2, 0, 1)
